# baseline (speedup 1.0000x reference)
import functools

import jax
import jax.numpy as jnp
from jax import lax
from jax.experimental import pallas as pl
from jax.experimental.pallas import tpu as pltpu
from jax.experimental.pallas import tpu_sc as plsc

NUM_CORES = 2
NUM_SUBCORES = 16
NUM_TILES = NUM_CORES * NUM_SUBCORES
CHUNK = 128


NBUF = 4


def _make_spmm(n_pad, d, n_chunks):
  assert n_chunks % NBUF == 0
  rows_per_tile = n_pad // NUM_SUBCORES
  tile_edges = n_chunks * CHUNK
  mesh = plsc.VectorSubcoreMesh(core_axis_name="c", subcore_axis_name="s")

  @functools.partial(
      pl.kernel,
      out_type=jax.ShapeDtypeStruct((NUM_CORES, n_pad, d), jnp.float32),
      mesh=mesh,
      compiler_params=pltpu.CompilerParams(use_tc_tiling_on_sc=False),
      scratch_types=[
          pltpu.VMEM_SHARED((n_pad, d), jnp.float32),
          pltpu.VMEM((tile_edges,), jnp.int32),
          pltpu.VMEM((n_chunks, 1, CHUNK), jnp.int32),
          [pltpu.VMEM((CHUNK, d), jnp.float32) for _ in range(NBUF)],
          [pltpu.SemaphoreType.DMA for _ in range(NBUF)],
          [pltpu.SemaphoreType.DMA for _ in range(NBUF)],
          pltpu.SemaphoreType.DMA,
      ],
  )
  def spmm(src_hbm, dst_hbm, v_hbm, zeros_hbm, out_hbm,
           acc, src_v, dst_v, bufs, gsems, ssems, sem):
    c = lax.axis_index("c")
    s = lax.axis_index("s")
    row0 = s * rows_per_tile
    wid = c * NUM_SUBCORES + s
    base = wid * tile_edges

    pltpu.async_copy(src_hbm.at[pl.ds(base, tile_edges)], src_v, sem)
    pltpu.async_copy(dst_hbm.at[wid], dst_v, sem)
    pltpu.sync_copy(zeros_hbm.at[pl.ds(row0, rows_per_tile)],
                    acc.at[pl.ds(row0, rows_per_tile)])
    pltpu.make_async_copy(src_hbm.at[pl.ds(base, tile_edges)], src_v,
                          sem).wait()
    pltpu.make_async_copy(dst_hbm.at[wid], dst_v, sem).wait()
    plsc.subcore_barrier()

    def gather(i, k):
      pltpu.async_copy(v_hbm.at[src_v.at[pl.ds(i * CHUNK, CHUNK)]],
                       bufs[k], gsems[k])

    def gather_wait(k):
      pltpu.make_async_copy(v_hbm.at[src_v.at[pl.ds(0, CHUNK)]],
                            bufs[k], gsems[k]).wait()

    def scatter(i, k):
      pltpu.async_copy(bufs[k], acc.at[dst_v.at[i, 0]], ssems[k], add=True)

    def scatter_wait(k):
      pltpu.make_async_copy(bufs[k], acc.at[dst_v.at[0, 0]], ssems[k]).wait()

    for k in range(NBUF):
      gather(k, k)

    def body(j, _):
      for k in range(NBUF):
        gather_wait(k)
        scatter(j * NBUF + k, k)
      for k in range(NBUF):
        scatter_wait(k)
        gather((j + 1) * NBUF + k, k)
      return 0

    lax.fori_loop(0, n_chunks // NBUF - 1, body, 0)
    for k in range(NBUF):
      gather_wait(k)
      scatter((n_chunks // NBUF - 1) * NBUF + k, k)
    for k in range(NBUF):
      scatter_wait(k)

    plsc.subcore_barrier()
    pltpu.sync_copy(acc.at[pl.ds(row0, rows_per_tile)],
                    out_hbm.at[c, pl.ds(row0, rows_per_tile)])

  return spmm


def _mm_split(pa, pb, wa, wb, b):
  def body(pa_ref, pb_ref, wa_ref, wb_ref, b_ref, o_ref):
    ya = jnp.dot(pa_ref[0] + pa_ref[1], wa_ref[...],
                 preferred_element_type=jnp.float32)
    yb = jnp.dot(pb_ref[0] + pb_ref[1], wb_ref[...],
                 preferred_element_type=jnp.float32)
    o_ref[...] = jnp.maximum(ya + yb + b_ref[...], 0.0)
  return pl.pallas_call(
      body,
      out_shape=jax.ShapeDtypeStruct((pa.shape[1], wa.shape[1]),
                                     jnp.float32),
  )(pa, pb, wa, wb, b)


def _mm_fused(parts, wt, b, activation):
  def body(p_ref, w_ref, b_ref, o_ref):
    h = p_ref[0] + p_ref[1]
    if activation == "relu":
      y = jnp.dot(h, w_ref[...], preferred_element_type=jnp.float32)
      y = jnp.maximum(y + b_ref[...], 0.0)
    else:
      y = jnp.dot(h, w_ref[...], preferred_element_type=jnp.float32,
                  precision=jax.lax.Precision.HIGHEST)
      y = jax.nn.softplus(y + b_ref[...])
    o_ref[...] = y
  return pl.pallas_call(
      body,
      out_shape=jax.ShapeDtypeStruct((parts.shape[1], wt.shape[1]),
                                     jnp.float32),
  )(parts, wt, b)


def kernel(x, edge_index, W1, b1, W2, b2, W_out, b_out):
  n, d_in = x.shape
  e = edge_index.shape[1]
  d_h = W1.shape[0]

  align = NUM_SUBCORES * 8
  n_pad = ((n + align - 1) // align) * align
  if n_pad == n:
    n_pad += align
  edges_per_super = NUM_TILES * CHUNK
  n_chunks = (e + edges_per_super - 1) // edges_per_super
  n_chunks = ((n_chunks + NBUF - 1) // NBUF) * NBUF
  e_pad = n_chunks * edges_per_super

  src = jnp.concatenate(
      [edge_index[0], jnp.zeros((e_pad - e,), jnp.int32)])
  pad_dst = (n + jnp.arange(e_pad - e, dtype=jnp.int32) % (n_pad - n))
  dst = jnp.concatenate([edge_index[1], pad_dst])
  dst = dst.reshape(NUM_TILES, n_chunks, 1, CHUNK)

  zeros_h = jnp.zeros((n_pad, d_h), jnp.float32)

  wt1 = W1.T
  wt2 = W2.T
  wt_out = jnp.pad(W_out, ((0, 15), (0, 0))).T
  b1r = b1.reshape(1, d_h)
  b2r = b2.reshape(1, d_h)
  b_out_r = jnp.broadcast_to(b_out.reshape(1, 1), (1, 16))

  spmm_h = _make_spmm(n_pad, d_h, n_chunks)

  half = d_in // 2
  p1a = spmm_h(src, dst, x[:, :half], zeros_h)
  p1b = spmm_h(src, dst, x[:, half:], zeros_h)
  h1 = _mm_split(p1a, p1b, wt1[:half], wt1[half:], b1r)
  p2 = spmm_h(src, dst, h1, zeros_h)
  h2 = _mm_fused(p2, wt2, b2r, "relu")
  p3 = spmm_h(src, dst, h2, zeros_h)
  out16 = _mm_fused(p3, wt_out, b_out_r, "softplus")
  return out16[:n, :1]


# speedup vs baseline: 1.0179x; 1.0179x over previous
import functools

import jax
import jax.numpy as jnp
from jax import lax
from jax.experimental import pallas as pl
from jax.experimental.pallas import tpu as pltpu
from jax.experimental.pallas import tpu_sc as plsc

NUM_CORES = 2
NUM_SUBCORES = 16
NUM_TILES = NUM_CORES * NUM_SUBCORES
CHUNK = 128


NBUF = 8


def _make_spmm(n_pad, d, n_chunks):
  assert n_chunks % NBUF == 0
  rows_per_tile = n_pad // NUM_SUBCORES
  tile_edges = n_chunks * CHUNK
  mesh = plsc.VectorSubcoreMesh(core_axis_name="c", subcore_axis_name="s")

  @functools.partial(
      pl.kernel,
      out_type=jax.ShapeDtypeStruct((NUM_CORES, n_pad, d), jnp.float32),
      mesh=mesh,
      compiler_params=pltpu.CompilerParams(use_tc_tiling_on_sc=False),
      scratch_types=[
          pltpu.VMEM_SHARED((n_pad, d), jnp.float32),
          pltpu.VMEM((tile_edges,), jnp.int32),
          pltpu.VMEM((n_chunks, 1, CHUNK), jnp.int32),
          [pltpu.VMEM((CHUNK, d), jnp.float32) for _ in range(NBUF)],
          [pltpu.SemaphoreType.DMA for _ in range(NBUF)],
          [pltpu.SemaphoreType.DMA for _ in range(NBUF)],
          pltpu.SemaphoreType.DMA,
      ],
  )
  def spmm(src_hbm, dst_hbm, v_hbm, zeros_hbm, out_hbm,
           acc, src_v, dst_v, bufs, gsems, ssems, sem):
    c = lax.axis_index("c")
    s = lax.axis_index("s")
    row0 = s * rows_per_tile
    wid = c * NUM_SUBCORES + s
    base = wid * tile_edges

    pltpu.async_copy(src_hbm.at[pl.ds(base, tile_edges)], src_v, sem)
    pltpu.async_copy(dst_hbm.at[wid], dst_v, sem)
    pltpu.sync_copy(zeros_hbm.at[pl.ds(row0, rows_per_tile)],
                    acc.at[pl.ds(row0, rows_per_tile)])
    pltpu.make_async_copy(src_hbm.at[pl.ds(base, tile_edges)], src_v,
                          sem).wait()
    pltpu.make_async_copy(dst_hbm.at[wid], dst_v, sem).wait()
    plsc.subcore_barrier()

    def gather(i, k):
      pltpu.async_copy(v_hbm.at[src_v.at[pl.ds(i * CHUNK, CHUNK)]],
                       bufs[k], gsems[k])

    def gather_wait(k):
      pltpu.make_async_copy(v_hbm.at[src_v.at[pl.ds(0, CHUNK)]],
                            bufs[k], gsems[k]).wait()

    def scatter(i, k):
      pltpu.async_copy(bufs[k], acc.at[dst_v.at[i, 0]], ssems[k], add=True)

    def scatter_wait(k):
      pltpu.make_async_copy(bufs[k], acc.at[dst_v.at[0, 0]], ssems[k]).wait()

    for k in range(NBUF):
      gather(k, k)

    def body(j, _):
      for k in range(NBUF):
        gather_wait(k)
        scatter(j * NBUF + k, k)
      for k in range(NBUF):
        scatter_wait(k)
        gather((j + 1) * NBUF + k, k)
      return 0

    lax.fori_loop(0, n_chunks // NBUF - 1, body, 0)
    for k in range(NBUF):
      gather_wait(k)
      scatter((n_chunks // NBUF - 1) * NBUF + k, k)
    for k in range(NBUF):
      scatter_wait(k)

    plsc.subcore_barrier()
    pltpu.sync_copy(acc.at[pl.ds(row0, rows_per_tile)],
                    out_hbm.at[c, pl.ds(row0, rows_per_tile)])

  return spmm


def _mm_split(pa, pb, wa, wb, b):
  def body(pa_ref, pb_ref, wa_ref, wb_ref, b_ref, o_ref):
    ya = jnp.dot(pa_ref[0] + pa_ref[1], wa_ref[...],
                 preferred_element_type=jnp.float32)
    yb = jnp.dot(pb_ref[0] + pb_ref[1], wb_ref[...],
                 preferred_element_type=jnp.float32)
    o_ref[...] = jnp.maximum(ya + yb + b_ref[...], 0.0)
  return pl.pallas_call(
      body,
      out_shape=jax.ShapeDtypeStruct((pa.shape[1], wa.shape[1]),
                                     jnp.float32),
  )(pa, pb, wa, wb, b)


def _mm_fused(parts, wt, b, activation):
  def body(p_ref, w_ref, b_ref, o_ref):
    h = p_ref[0] + p_ref[1]
    if activation == "relu":
      y = jnp.dot(h, w_ref[...], preferred_element_type=jnp.float32)
      y = jnp.maximum(y + b_ref[...], 0.0)
    else:
      y = jnp.dot(h, w_ref[...], preferred_element_type=jnp.float32,
                  precision=jax.lax.Precision.HIGHEST)
      y = jax.nn.softplus(y + b_ref[...])
    o_ref[...] = y
  return pl.pallas_call(
      body,
      out_shape=jax.ShapeDtypeStruct((parts.shape[1], wt.shape[1]),
                                     jnp.float32),
  )(parts, wt, b)


def kernel(x, edge_index, W1, b1, W2, b2, W_out, b_out):
  n, d_in = x.shape
  e = edge_index.shape[1]
  d_h = W1.shape[0]

  align = NUM_SUBCORES * 8
  n_pad = ((n + align - 1) // align) * align
  if n_pad == n:
    n_pad += align
  edges_per_super = NUM_TILES * CHUNK
  n_chunks = (e + edges_per_super - 1) // edges_per_super
  n_chunks = ((n_chunks + NBUF - 1) // NBUF) * NBUF
  e_pad = n_chunks * edges_per_super

  src = jnp.concatenate(
      [edge_index[0], jnp.zeros((e_pad - e,), jnp.int32)])
  pad_dst = (n + jnp.arange(e_pad - e, dtype=jnp.int32) % (n_pad - n))
  dst = jnp.concatenate([edge_index[1], pad_dst])
  dst = dst.reshape(NUM_TILES, n_chunks, 1, CHUNK)

  zeros_h = jnp.zeros((n_pad, d_h), jnp.float32)

  wt1 = W1.T
  wt2 = W2.T
  wt_out = jnp.pad(W_out, ((0, 15), (0, 0))).T
  b1r = b1.reshape(1, d_h)
  b2r = b2.reshape(1, d_h)
  b_out_r = jnp.broadcast_to(b_out.reshape(1, 1), (1, 16))

  spmm_h = _make_spmm(n_pad, d_h, n_chunks)

  half = d_in // 2
  p1a = spmm_h(src, dst, x[:, :half], zeros_h)
  p1b = spmm_h(src, dst, x[:, half:], zeros_h)
  h1 = _mm_split(p1a, p1b, wt1[:half], wt1[half:], b1r)
  p2 = spmm_h(src, dst, h1, zeros_h)
  h2 = _mm_fused(p2, wt2, b2r, "relu")
  p3 = spmm_h(src, dst, h2, zeros_h)
  out16 = _mm_fused(p3, wt_out, b_out_r, "softplus")
  return out16[:n, :1]


# speedup vs baseline: 1.0554x; 1.0368x over previous
import functools

import jax
import jax.numpy as jnp
from jax import lax
from jax.experimental import pallas as pl
from jax.experimental.pallas import tpu as pltpu
from jax.experimental.pallas import tpu_sc as plsc

NUM_CORES = 2
NUM_SUBCORES = 16
NUM_TILES = NUM_CORES * NUM_SUBCORES
CHUNK = 128


NBUF = 4
NC_FAST = 128
NC_SLOW = 32


def _make_spmm(n_pad, d):
  rows_per_tile = n_pad // NUM_SUBCORES
  mesh = plsc.VectorSubcoreMesh(core_axis_name="c", subcore_axis_name="s")

  @functools.partial(
      pl.kernel,
      out_type=jax.ShapeDtypeStruct((NUM_CORES, n_pad, d), jnp.float32),
      mesh=mesh,
      compiler_params=pltpu.CompilerParams(use_tc_tiling_on_sc=False),
      scratch_types=[
          pltpu.VMEM_SHARED((n_pad, d), jnp.float32),
          pltpu.VMEM((NC_FAST * CHUNK,), jnp.int32),
          pltpu.VMEM((NC_FAST, 1, CHUNK), jnp.int32),
          [pltpu.VMEM((CHUNK, d), jnp.float32) for _ in range(NBUF)],
          [pltpu.SemaphoreType.DMA for _ in range(NBUF)],
          [pltpu.SemaphoreType.DMA for _ in range(NBUF)],
          pltpu.SemaphoreType.DMA,
      ],
  )
  def spmm(src_hbm, dst_hbm, v_hbm, zeros_hbm, out_hbm,
           acc, src_v, dst_v, bufs, gsems, ssems, sem):
    c = lax.axis_index("c")
    s = lax.axis_index("s")
    row0 = s * rows_per_tile
    base_fast = s * NC_FAST
    base_slow = NUM_SUBCORES * NC_FAST + s * NC_SLOW

    @pl.when(c == 0)
    def _():
      pltpu.async_copy(
          src_hbm.at[pl.ds(base_fast * CHUNK, NC_FAST * CHUNK)], src_v, sem)
      pltpu.async_copy(dst_hbm.at[pl.ds(base_fast, NC_FAST)], dst_v, sem)

    @pl.when(c == 1)
    def _():
      pltpu.async_copy(
          src_hbm.at[pl.ds(base_slow * CHUNK, NC_SLOW * CHUNK)],
          src_v.at[pl.ds(0, NC_SLOW * CHUNK)], sem)
      pltpu.async_copy(dst_hbm.at[pl.ds(base_slow, NC_SLOW)],
                       dst_v.at[pl.ds(0, NC_SLOW)], sem)

    pltpu.sync_copy(zeros_hbm.at[pl.ds(row0, rows_per_tile)],
                    acc.at[pl.ds(row0, rows_per_tile)])

    @pl.when(c == 0)
    def _():
      pltpu.make_async_copy(
          src_hbm.at[pl.ds(base_fast * CHUNK, NC_FAST * CHUNK)], src_v,
          sem).wait()
      pltpu.make_async_copy(dst_hbm.at[pl.ds(base_fast, NC_FAST)], dst_v,
                            sem).wait()

    @pl.when(c == 1)
    def _():
      pltpu.make_async_copy(
          src_hbm.at[pl.ds(base_slow * CHUNK, NC_SLOW * CHUNK)],
          src_v.at[pl.ds(0, NC_SLOW * CHUNK)], sem).wait()
      pltpu.make_async_copy(dst_hbm.at[pl.ds(base_slow, NC_SLOW)],
                            dst_v.at[pl.ds(0, NC_SLOW)], sem).wait()

    plsc.subcore_barrier()

    def gather(i, k):
      pltpu.async_copy(v_hbm.at[src_v.at[pl.ds(i * CHUNK, CHUNK)]],
                       bufs[k], gsems[k])

    def gather_wait(k):
      pltpu.make_async_copy(v_hbm.at[src_v.at[pl.ds(0, CHUNK)]],
                            bufs[k], gsems[k]).wait()

    def scatter(i, k):
      pltpu.async_copy(bufs[k], acc.at[dst_v.at[i, 0]], ssems[k], add=True)

    def scatter_wait(k):
      pltpu.make_async_copy(bufs[k], acc.at[dst_v.at[0, 0]], ssems[k]).wait()

    my_nc = jnp.where(c == 0, NC_FAST, NC_SLOW)
    n_iters = my_nc // NBUF

    for k in range(NBUF):
      gather(k, k)

    def body(j, _):
      for k in range(NBUF):
        gather_wait(k)
        scatter(j * NBUF + k, k)
      for k in range(NBUF):
        scatter_wait(k)
        gather((j + 1) * NBUF + k, k)
      return 0

    lax.fori_loop(0, n_iters - 1, body, 0)
    for k in range(NBUF):
      gather_wait(k)
      scatter((n_iters - 1) * NBUF + k, k)
    for k in range(NBUF):
      scatter_wait(k)

    plsc.subcore_barrier()
    pltpu.sync_copy(acc.at[pl.ds(row0, rows_per_tile)],
                    out_hbm.at[c, pl.ds(row0, rows_per_tile)])

  return spmm


def _mm_split(pa, pb, wa, wb, b):
  def body(pa_ref, pb_ref, wa_ref, wb_ref, b_ref, o_ref):
    ya = jnp.dot(pa_ref[0] + pa_ref[1], wa_ref[...],
                 preferred_element_type=jnp.float32)
    yb = jnp.dot(pb_ref[0] + pb_ref[1], wb_ref[...],
                 preferred_element_type=jnp.float32)
    o_ref[...] = jnp.maximum(ya + yb + b_ref[...], 0.0)
  return pl.pallas_call(
      body,
      out_shape=jax.ShapeDtypeStruct((pa.shape[1], wa.shape[1]),
                                     jnp.float32),
  )(pa, pb, wa, wb, b)


def _mm_fused(parts, wt, b, activation):
  def body(p_ref, w_ref, b_ref, o_ref):
    h = p_ref[0] + p_ref[1]
    if activation == "relu":
      y = jnp.dot(h, w_ref[...], preferred_element_type=jnp.float32)
      y = jnp.maximum(y + b_ref[...], 0.0)
    else:
      y = jnp.dot(h, w_ref[...], preferred_element_type=jnp.float32,
                  precision=jax.lax.Precision.HIGHEST)
      y = jax.nn.softplus(y + b_ref[...])
    o_ref[...] = y
  return pl.pallas_call(
      body,
      out_shape=jax.ShapeDtypeStruct((parts.shape[1], wt.shape[1]),
                                     jnp.float32),
  )(parts, wt, b)


def kernel(x, edge_index, W1, b1, W2, b2, W_out, b_out):
  n, d_in = x.shape
  e = edge_index.shape[1]
  d_h = W1.shape[0]

  align = NUM_SUBCORES * 8
  n_pad = ((n + align - 1) // align) * align
  if n_pad == n:
    n_pad += align
  total_chunks = NUM_SUBCORES * (NC_FAST + NC_SLOW)
  e_pad = total_chunks * CHUNK
  assert e <= e_pad

  src = jnp.concatenate(
      [edge_index[0], jnp.zeros((e_pad - e,), jnp.int32)])
  pad_dst = (n + jnp.arange(e_pad - e, dtype=jnp.int32) % (n_pad - n))
  dst = jnp.concatenate([edge_index[1], pad_dst])
  dst = dst.reshape(total_chunks, 1, CHUNK)

  zeros_h = jnp.zeros((n_pad, d_h), jnp.float32)

  wt1 = W1.T
  wt2 = W2.T
  wt_out = jnp.pad(W_out, ((0, 15), (0, 0))).T
  b1r = b1.reshape(1, d_h)
  b2r = b2.reshape(1, d_h)
  b_out_r = jnp.broadcast_to(b_out.reshape(1, 1), (1, 16))

  spmm_h = _make_spmm(n_pad, d_h)

  half = d_in // 2
  p1a = spmm_h(src, dst, x[:, :half], zeros_h)
  p1b = spmm_h(src, dst, x[:, half:], zeros_h)
  h1 = _mm_split(p1a, p1b, wt1[:half], wt1[half:], b1r)
  p2 = spmm_h(src, dst, h1, zeros_h)
  h2 = _mm_fused(p2, wt2, b2r, "relu")
  p3 = spmm_h(src, dst, h2, zeros_h)
  out16 = _mm_fused(p3, wt_out, b_out_r, "softplus")
  return out16[:n, :1]
